# baseline (speedup 1.0000x reference)
import functools

import jax
import jax.numpy as jnp
from jax import lax
from jax.experimental import pallas as pl
from jax.experimental.pallas import tpu as pltpu
from jax.experimental.pallas import tpu_sc as plsc

_T = 2048
_S = 2048
_E = 16
_ROW = _S * _E
_CLS = 65536
_NSH = 8
_NC = 2
_ROWS_PER_C = _T // _NC

_smesh = plsc.ScalarSubcoreMesh(axis_name="c", num_cores=_NC)


@functools.partial(
    pl.kernel,
    mesh=_smesh,
    out_type=jax.ShapeDtypeStruct((_T * _ROW,), jnp.float32),
    scratch_types=[
        pltpu.VMEM_SHARED((_NSH * _CLS,), jnp.float32),
        pltpu.SemaphoreType.DMA,
    ],
)
def _scs_window_copy(shifted_hbm, out_hbm, table_s, sem):
    cid = lax.axis_index("c")
    pltpu.sync_copy(shifted_hbm, table_s)
    base = cid * _ROWS_PER_C

    def body(r, carry):
        i = base + r
        n = _T - 1 - i
        b = n % _NSH
        a = n // _NSH
        pltpu.sync_copy(
            table_s.at[pl.ds(b * _CLS + 128 * a, _ROW)],
            out_hbm.at[pl.ds(i * _ROW, _ROW)],
        )
        return carry

    lax.fori_loop(0, _ROWS_PER_C, body, 0)


def kernel(table):
    flat = table.reshape(-1)
    padded = jnp.pad(flat, (0, _CLS - flat.shape[0]))
    shifted = jnp.stack(
        [lax.dynamic_slice_in_dim(padded, _E * b, _CLS - 128) for b in range(_NSH)]
    )
    shifted = jnp.pad(shifted, ((0, 0), (0, 128))).reshape(-1)
    out = _scs_window_copy(shifted)
    return out.reshape(_T, _S, _E)


# speedup vs baseline: 1.3017x; 1.3017x over previous
import functools

import jax
import jax.numpy as jnp
from jax import lax
from jax.experimental import pallas as pl
from jax.experimental.pallas import tpu as pltpu
from jax.experimental.pallas import tpu_sc as plsc

_T = 2048
_S = 2048
_E = 16
_ROW = _S * _E
_CLS = 65536
_NSH = 8
_NC = 2
_ROWS_PER_C = _T // _NC
_K = 16

_smesh = plsc.ScalarSubcoreMesh(axis_name="c", num_cores=_NC)


@functools.partial(
    pl.kernel,
    mesh=_smesh,
    out_type=jax.ShapeDtypeStruct((_T * _ROW,), jnp.float32),
    scratch_types=[
        pltpu.VMEM_SHARED((_NSH * _CLS,), jnp.float32),
        pltpu.SemaphoreType.DMA,
    ],
)
def _scs_window_copy(shifted_hbm, out_hbm, table_s, sem):
    cid = lax.axis_index("c")
    pltpu.sync_copy(shifted_hbm, table_s)
    base = cid * _ROWS_PER_C

    def body(g, carry):
        i0 = base + g * _K
        descs = []
        for r in range(_K):
            i = i0 + r
            n = _T - 1 - i
            b = n % _NSH
            a = n // _NSH
            d = pltpu.make_async_copy(
                table_s.at[pl.ds(b * _CLS + 128 * a, _ROW)],
                out_hbm.at[pl.ds(i * _ROW, _ROW)],
                sem,
            )
            d.start()
            descs.append(d)
        for d in descs:
            d.wait()
        return carry

    lax.fori_loop(0, _ROWS_PER_C // _K, body, 0)


def kernel(table):
    flat = table.reshape(-1)
    padded = jnp.pad(flat, (0, _CLS - flat.shape[0]))
    shifted = jnp.stack(
        [lax.dynamic_slice_in_dim(padded, _E * b, _CLS - 128) for b in range(_NSH)]
    )
    shifted = jnp.pad(shifted, ((0, 0), (0, 128))).reshape(-1)
    out = _scs_window_copy(shifted)
    return out.reshape(_T, _S, _E)


# speedup vs baseline: 1.3928x; 1.0700x over previous
import functools

import jax
import jax.numpy as jnp
from jax import lax
from jax.experimental import pallas as pl
from jax.experimental.pallas import tpu as pltpu
from jax.experimental.pallas import tpu_sc as plsc

_T = 2048
_S = 2048
_E = 16
_FLAT = (_T + _S - 1) * _E
_ROW = _S * _E
_CLS = 65536
_NSH = 8
_NC = 2
_NS = 16
_NW = _NC * _NS
_RPW = _T // _NW

_mesh = plsc.VectorSubcoreMesh(core_axis_name="c", subcore_axis_name="s")


@functools.partial(
    pl.kernel,
    mesh=_mesh,
    out_type=jax.ShapeDtypeStruct((_T * _ROW,), jnp.float32),
    scratch_types=[
        pltpu.VMEM((_FLAT,), jnp.float32),
        pltpu.VMEM_SHARED((_NSH * _CLS,), jnp.float32),
        pltpu.SemaphoreType.DMA,
        pltpu.SemaphoreType.DMA,
    ],
)
def _sc_window_copy(flat_hbm, shifted_hbm, out_hbm, flat_v, shifted_s, sem_st, sem_dma):
    wid = lax.axis_index("s") * _NC + lax.axis_index("c")
    @pl.when(lax.axis_index("s") == 0)
    def _stage_shared():
        pltpu.sync_copy(shifted_hbm, shifted_s)

    pltpu.sync_copy(flat_hbm, flat_v)
    plsc.subcore_barrier()

    base = wid * _RPW

    def body(g, carry):
        descs = []
        for r in range(2):
            i = base + g * 2 + r
            n = _T - 1 - i
            if r == 0:
                d = pltpu.make_async_copy(
                    flat_v.at[pl.ds(n * _E, _ROW)],
                    out_hbm.at[pl.ds(i * _ROW, _ROW)],
                    sem_st,
                )
            else:
                d = pltpu.make_async_copy(
                    shifted_s.at[pl.ds((n % _NSH) * _CLS + 128 * (n // _NSH), _ROW)],
                    out_hbm.at[pl.ds(i * _ROW, _ROW)],
                    sem_dma,
                )
            d.start()
            descs.append(d)
        for d in descs:
            d.wait()
        return carry

    lax.fori_loop(0, _RPW // 2, body, 0)


def kernel(table):
    flat = table.reshape(-1)
    padded = jnp.pad(flat, (0, _CLS - flat.shape[0]))
    shifted = jnp.stack(
        [lax.dynamic_slice_in_dim(padded, _E * b, _CLS - 128) for b in range(_NSH)]
    )
    shifted = jnp.pad(shifted, ((0, 0), (0, 128))).reshape(-1)
    out = _sc_window_copy(flat, shifted)
    return out.reshape(_T, _S, _E)


# speedup vs baseline: 3.9693x; 2.8499x over previous
import jax
import jax.numpy as jnp
from jax import lax
from jax.experimental import pallas as pl
from jax.experimental.pallas import tpu as pltpu

_T = 2048
_S = 2048
_E = 16
_RB = 64
_G = _T // _RB
_PAD = 16 * 63 + 65536


def _tc_body(sh_ref, out_ref):
    g = pl.program_id(0)
    off = 8 * (_G - 1 - g)
    for r in range(_RB):
        out_ref[r] = sh_ref[_RB - 1 - r, pl.ds(off, 256), :]


def kernel(table):
    flat = table.reshape(-1)
    padded = jnp.pad(flat, (0, _PAD - flat.shape[0]))
    shifted64 = jnp.stack(
        [lax.dynamic_slice_in_dim(padded, _E * b, 65536) for b in range(64)]
    ).reshape(64, 512, 128)
    out = pl.pallas_call(
        _tc_body,
        grid=(_G,),
        in_specs=[pl.BlockSpec((64, 512, 128), lambda g: (0, 0, 0))],
        out_specs=pl.BlockSpec((_RB, 256, 128), lambda g: (g, 0, 0)),
        out_shape=jax.ShapeDtypeStruct((_T, 256, 128), jnp.float32),
    )(shifted64)
    return out.reshape(_T, _S, _E)


# speedup vs baseline: 4.3938x; 1.1069x over previous
import jax
import jax.numpy as jnp
from jax.experimental import pallas as pl
from jax.experimental.pallas import tpu as pltpu

_T = 2048
_S = 2048
_E = 16
_RB = 64
_G = _T // _RB
_PADROWS = 520


def _tc_body(f_ref, out_ref, sh_ref):
    g = pl.program_id(0)

    @pl.when(g == 0)
    def _build():
        f = f_ref[...]
        for b in range(64):
            rb, lb = b // 8, 16 * (b % 8)
            if lb == 0:
                sh_ref[b] = f[rb:rb + 512, :]
            else:
                sh_ref[b] = jnp.concatenate(
                    [f[rb:rb + 512, lb:], f[rb + 1:rb + 513, :lb]], axis=1
                )

    off = 8 * (_G - 1 - g)
    for r in range(_RB):
        out_ref[r] = sh_ref[_RB - 1 - r, pl.ds(off, 256), :]


def kernel(table):
    flat = table.reshape(-1)
    padded = jnp.pad(flat, (0, _PADROWS * 128 - flat.shape[0])).reshape(_PADROWS, 128)
    out = pl.pallas_call(
        _tc_body,
        grid=(_G,),
        in_specs=[pl.BlockSpec((_PADROWS, 128), lambda g: (0, 0))],
        out_specs=pl.BlockSpec((_RB, 256, 128), lambda g: (g, 0, 0)),
        out_shape=jax.ShapeDtypeStruct((_T, 256, 128), jnp.float32),
        scratch_shapes=[pltpu.VMEM((64, 512, 128), jnp.float32)],
    )(padded)
    return out.reshape(_T, _S, _E)
